# Initial kernel scaffold; baseline (speedup 1.0000x reference)
import jax
import jax.numpy as jnp
from jax.experimental import pallas as pl


def kernel(points, sa1, sa2, sa3, head):
    raise NotImplementedError("write your pallas kernel here")


# speedup vs baseline: 9.0774x; 9.0774x over previous
import functools

import jax
import jax.numpy as jnp
from jax import lax
from jax.experimental import pallas as pl

F32 = jnp.float32
BF16 = jnp.bfloat16
HI = lax.Precision.HIGHEST


def _bnorm(h, g, be):
    return h / jnp.sqrt(jnp.asarray(1.0 + 1e-5, F32)) * g + be


def _mm(a, b):
    return jnp.dot(a.astype(BF16), b.astype(BF16),
                   preferred_element_type=F32)


def _fps_body(x_ref, y_ref, z_ref, cx_ref, cy_ref, cz_ref, *, npoint):
    x = x_ref[...]
    y = y_ref[...]
    z = z_ref[...]
    B, N = x.shape
    iota = lax.broadcasted_iota(jnp.int32, (B, N), 1)
    iop = lax.broadcasted_iota(jnp.int32, (B, npoint), 1)

    def body(i, st):
        dist, far, ox, oy, oz = st
        sel = iota == far
        cx = jnp.sum(jnp.where(sel, x, 0.0), axis=1, keepdims=True)
        cy = jnp.sum(jnp.where(sel, y, 0.0), axis=1, keepdims=True)
        cz = jnp.sum(jnp.where(sel, z, 0.0), axis=1, keepdims=True)
        onec = iop == i
        ox = jnp.where(onec, cx, ox)
        oy = jnp.where(onec, cy, oy)
        oz = jnp.where(onec, cz, oz)
        d = (x - cx) ** 2 + (y - cy) ** 2 + (z - cz) ** 2
        dist = jnp.minimum(dist, d)
        m = jnp.max(dist, axis=1, keepdims=True)
        far = jnp.min(jnp.where(dist == m, iota, N), axis=1, keepdims=True)
        return dist, far, ox, oy, oz

    st = (jnp.full((B, N), 1e10, F32), jnp.zeros((B, 1), jnp.int32),
          jnp.zeros((B, npoint), F32), jnp.zeros((B, npoint), F32),
          jnp.zeros((B, npoint), F32))
    _, _, ox, oy, oz = lax.fori_loop(0, npoint, body, st)
    cx_ref[...] = ox
    cy_ref[...] = oy
    cz_ref[...] = oz


def _fps(x, y, z, npoint):
    B, N = x.shape
    out = jax.ShapeDtypeStruct((B, npoint), F32)
    return pl.pallas_call(
        functools.partial(_fps_body, npoint=npoint),
        out_shape=(out, out, out),
    )(x, y, z)


def _rank_of(mask_f, S, N):
    io = lax.broadcasted_iota(jnp.int32, (128, 128), 0)
    jo = lax.broadcasted_iota(jnp.int32, (128, 128), 1)
    tri = (io <= jo).astype(F32)
    parts = []
    off = jnp.zeros((S, 1), F32)
    for c in range(N // 128):
        mc = mask_f[:, c * 128:(c + 1) * 128]
        rc = jnp.dot(mc, tri, precision=HI) + off
        off = rc[:, 127:128]
        parts.append(rc)
    return jnp.concatenate(parts, axis=1)


def _sa_body(x_ref, y_ref, z_ref, cxc_ref, cyc_ref, czc_ref, pts_ref,
             ptsT_ref, ctr_ref, f_ref, w1p_ref, w1f_ref, b1_ref, g1_ref,
             be1_ref, w2_ref, b2_ref, g2_ref, be2_ref,
             w3_ref, b3_ref, g3_ref, be3_ref, out_ref,
             *, S, N, ns, r2):
    x = x_ref[0]
    y = y_ref[0]
    z = z_ref[0]
    cx = cxc_ref[...].reshape(S, 1)
    cy = cyc_ref[...].reshape(S, 1)
    cz = czc_ref[...].reshape(S, 1)
    nc = cx * cx + cy * cy + cz * cz
    npt = x * x + y * y + z * z
    ctr = ctr_ref[0]
    M = _mm(ctr, ptsT_ref[0])
    sq = (nc + npt) - 2.0 * M
    mask_f = jnp.where(sq > r2, 0.0, 1.0)
    rank = _rank_of(mask_f, S, N)
    empty = rank[:, N - 1:N] < 0.5
    pts = pts_ref[0]
    have_f = f_ref is not None
    Ff = f_ref[0] if have_f else None

    w1p = w1p_ref[...]
    w1f = w1f_ref[...] if have_f else None
    b1 = b1_ref[...]
    g1 = g1_ref[...]
    be1 = be1_ref[...]
    w2 = w2_ref[...]
    b2 = b2_ref[...]
    g2 = g2_ref[...]
    be2 = be2_ref[...]
    w3 = w3_ref[...]
    b3 = b3_ref[...]
    g3 = g3_ref[...]
    be3 = be3_ref[...]

    def gather_k(kf):
        oh = jnp.where(rank == kf, mask_f, 0.0)
        g4 = jnp.dot(oh, pts, precision=HI)
        rel = g4 - ctr
        fk = jnp.dot(oh, Ff, precision=HI) if have_f else None
        return rel, fk

    def relu(v):
        return jnp.maximum(v, 0.0)

    def mlp(rel, fk):
        h = _mm(rel, w1p) + b1
        if have_f:
            h = h + _mm(fk, w1f)
        h = relu(_bnorm(h, g1, be1))
        h = relu(_bnorm(_mm(h, w2) + b2, g2, be2))
        h = relu(_bnorm(_mm(h, w3) + b3, g3, be3))
        return h

    rel_g, f_g = gather_k(1.0)
    rel0 = jnp.where(empty, pts[N - 1:N, :] - ctr, rel_g)
    if have_f:
        f0 = jnp.where(empty, Ff[N - 1:N, :], f_g)
    else:
        f0 = None
    acc0 = mlp(rel0, f0)

    def body(k, st):
        if have_f:
            acc, rel0, f0 = st
        else:
            acc, rel0 = st
            f0 = None
        kf = (k + 1).astype(F32)
        rel, fk = gather_k(kf)
        v = rel[:, 3:4] > 0.5
        rel = jnp.where(v, rel, rel0)
        if have_f:
            fk = jnp.where(v, fk, f0)
        acc = jnp.maximum(acc, mlp(rel, fk))
        return (acc, rel0, f0) if have_f else (acc, rel0)

    st = (acc0, rel0, f0) if have_f else (acc0, rel0)
    st = lax.fori_loop(1, ns, body, st)
    out_ref[0] = st[0]


def _drop_f(body):
    def wrapped(x, y, z, cxc, cyc, czc, pts, ptsT, ctr,
                w1p, b1, g1, be1, w2, b2, g2, be2, w3, b3, g3, be3, out):
        return body(x, y, z, cxc, cyc, czc, pts, ptsT, ctr, None,
                    w1p, None, b1, g1, be1, w2, b2, g2, be2,
                    w3, b3, g3, be3, out)
    return wrapped


def _sa_level(x, y, z, ncx, ncy, ncz, feats, layers, ns, radius):
    B, N = x.shape
    S = ncx.shape[1]
    (W1, b1, g1, be1), (W2, b2, g2, be2), (W3, b3, g3, be3) = layers
    C3 = W3.shape[1]
    pts = jnp.stack([x, y, z, jnp.ones_like(x)], axis=-1)
    ptsT = jnp.stack([x, y, z, jnp.ones_like(x)], axis=1)
    ctr = jnp.stack([ncx, ncy, ncz, jnp.zeros_like(ncx)], -1)
    cxc = ncx[..., None]
    cyc = ncy[..., None]
    czc = ncz[..., None]
    w1p = jnp.concatenate([W1[:3], jnp.zeros((1, W1.shape[1]), F32)], 0)
    have_f = feats is not None
    w1f = W1[3:] if have_f else jnp.zeros((1, 1), F32)

    def row(v):
        return v.reshape(1, -1)

    body = functools.partial(_sa_body, S=S, N=N, ns=ns, r2=radius * radius)
    if not have_f:
        body = _drop_f(body)

    in_specs = [
        pl.BlockSpec((1, 1, N), lambda b: (b, 0, 0)),
        pl.BlockSpec((1, 1, N), lambda b: (b, 0, 0)),
        pl.BlockSpec((1, 1, N), lambda b: (b, 0, 0)),
        pl.BlockSpec((1, S, 1), lambda b: (b, 0, 0)),
        pl.BlockSpec((1, S, 1), lambda b: (b, 0, 0)),
        pl.BlockSpec((1, S, 1), lambda b: (b, 0, 0)),
        pl.BlockSpec((1, N, 4), lambda b: (b, 0, 0)),
        pl.BlockSpec((1, 4, N), lambda b: (b, 0, 0)),
        pl.BlockSpec((1, S, 4), lambda b: (b, 0, 0)),
    ]
    args = [x[:, None, :], y[:, None, :], z[:, None, :], cxc, cyc, czc,
            pts, ptsT, ctr]
    if have_f:
        C = feats.shape[2]
        in_specs.append(pl.BlockSpec((1, N, C), lambda b: (b, 0, 0)))
        args.append(feats)
    wspecs = []
    wargs = []
    for w in (w1p, w1f, row(b1), row(g1), row(be1),
              W2, row(b2), row(g2), row(be2),
              W3, row(b3), row(g3), row(be3)):
        if not have_f and w is w1f:
            continue
        wspecs.append(pl.BlockSpec(w.shape, lambda b, nd=w.ndim: (0,) * nd))
        wargs.append(w)

    return pl.pallas_call(
        body,
        grid=(B,),
        in_specs=in_specs + wspecs,
        out_specs=pl.BlockSpec((1, S, C3), lambda b: (b, 0, 0)),
        out_shape=jax.ShapeDtypeStruct((B, S, C3), F32),
    )(*args, *wargs)


def _sa3_body(c_ref, f_ref, w1p_ref, w1f_ref, b1_ref, g1_ref, be1_ref,
              w2_ref, b2_ref, g2_ref, be2_ref,
              w3_ref, b3_ref, g3_ref, be3_ref, out_ref, *, nb, P):
    c = c_ref[...].reshape(nb * P, 4)
    f = f_ref[...].reshape(nb * P, -1)

    def relu(v):
        return jnp.maximum(v, 0.0)

    h = _mm(c, w1p_ref[...]) + _mm(f, w1f_ref[...]) + b1_ref[...]
    h = relu(_bnorm(h, g1_ref[...], be1_ref[...]))
    h = relu(_bnorm(_mm(h, w2_ref[...]) + b2_ref[...],
                    g2_ref[...], be2_ref[...]))
    h = relu(_bnorm(_mm(h, w3_ref[...]) + b3_ref[...],
                    g3_ref[...], be3_ref[...]))
    C3 = h.shape[1]
    out_ref[...] = jnp.max(h.reshape(nb, P, C3), axis=1)


def _sa3(cx, cy, cz, feats, layers, nb=8):
    B, P = cx.shape
    (W1, b1, g1, be1), (W2, b2, g2, be2), (W3, b3, g3, be3) = layers
    C = feats.shape[2]
    C3 = W3.shape[1]
    c4 = jnp.stack([cx, cy, cz, jnp.zeros_like(cx)], -1)
    w1p = jnp.concatenate([W1[:3], jnp.zeros((1, W1.shape[1]), F32)], 0)
    w1f = W1[3:]

    def row(v):
        return v.reshape(1, -1)

    wlist = [w1p, w1f, row(b1), row(g1), row(be1), W2, row(b2), row(g2),
             row(be2), W3, row(b3), row(g3), row(be3)]
    wspecs = [pl.BlockSpec(w.shape, lambda b, nd=w.ndim: (0,) * nd)
              for w in wlist]
    return pl.pallas_call(
        functools.partial(_sa3_body, nb=nb, P=P),
        grid=(B // nb,),
        in_specs=[pl.BlockSpec((nb, P, 4), lambda b: (b, 0, 0)),
                  pl.BlockSpec((nb, P, C), lambda b: (b, 0, 0))] + wspecs,
        out_specs=pl.BlockSpec((nb, C3), lambda b: (b, 0)),
        out_shape=jax.ShapeDtypeStruct((B, C3), F32),
    )(c4, feats, *wlist)


def _head_body(x_ref, w1_ref, b1_ref, g1_ref, be1_ref,
               w2_ref, b2_ref, g2_ref, be2_ref, w3_ref, b3_ref, out_ref):
    def relu(v):
        return jnp.maximum(v, 0.0)

    h = relu(_bnorm(_mm(x_ref[...], w1_ref[...]) + b1_ref[...],
                    g1_ref[...], be1_ref[...]))
    h = relu(_bnorm(_mm(h, w2_ref[...]) + b2_ref[...],
                    g2_ref[...], be2_ref[...]))
    out_ref[...] = _mm(h, w3_ref[...]) + b3_ref[...]


def _head(x, head):
    (W1, b1, g1, be1, W2, b2, g2, be2, W3, b3) = head
    B = x.shape[0]

    def row(v):
        return v.reshape(1, -1)

    args = [x, W1, row(b1), row(g1), row(be1),
            W2, row(b2), row(g2), row(be2), W3, row(b3)]
    return pl.pallas_call(
        _head_body,
        out_shape=jax.ShapeDtypeStruct((B, W3.shape[1]), F32),
    )(*args)


def kernel(points, sa1, sa2, sa3, head):
    x = points[..., 0]
    y = points[..., 1]
    z = points[..., 2]
    c1x, c1y, c1z = _fps(x, y, z, 512)
    f1 = _sa_level(x, y, z, c1x, c1y, c1z, None, sa1, 32, 0.2)
    c2x, c2y, c2z = _fps(c1x, c1y, c1z, 128)
    f2 = _sa_level(c1x, c1y, c1z, c2x, c2y, c2z, f1, sa2, 64, 0.4)
    f3 = _sa3(c2x, c2y, c2z, f2, sa3)
    return _head(f3, head)


# speedup vs baseline: 13.5009x; 1.4873x over previous
import functools

import jax
import jax.numpy as jnp
from jax import lax
from jax.experimental import pallas as pl

F32 = jnp.float32
BF16 = jnp.bfloat16
HI = lax.Precision.HIGHEST


def _bnorm(h, g, be):
    return h / jnp.sqrt(jnp.asarray(1.0 + 1e-5, F32)) * g + be


def _mm(a, b):
    return jnp.dot(a.astype(BF16), b.astype(BF16),
                   preferred_element_type=F32)


def _fps_body(x_ref, y_ref, z_ref, cx_ref, cy_ref, cz_ref, *, npoint):
    x = x_ref[...]
    y = y_ref[...]
    z = z_ref[...]
    B, N = x.shape
    iota = lax.broadcasted_iota(jnp.int32, (B, N), 1)
    iop = lax.broadcasted_iota(jnp.int32, (B, npoint), 1)

    def body(i, st):
        dist, far, ox, oy, oz = st
        sel = iota == far
        cx = jnp.sum(jnp.where(sel, x, 0.0), axis=1, keepdims=True)
        cy = jnp.sum(jnp.where(sel, y, 0.0), axis=1, keepdims=True)
        cz = jnp.sum(jnp.where(sel, z, 0.0), axis=1, keepdims=True)
        onec = iop == i
        ox = jnp.where(onec, cx, ox)
        oy = jnp.where(onec, cy, oy)
        oz = jnp.where(onec, cz, oz)
        d = (x - cx) ** 2 + (y - cy) ** 2 + (z - cz) ** 2
        dist = jnp.minimum(dist, d)
        m = jnp.max(dist, axis=1, keepdims=True)
        far = jnp.min(jnp.where(dist == m, iota, N), axis=1, keepdims=True)
        return dist, far, ox, oy, oz

    st = (jnp.full((B, N), 1e10, F32), jnp.zeros((B, 1), jnp.int32),
          jnp.zeros((B, npoint), F32), jnp.zeros((B, npoint), F32),
          jnp.zeros((B, npoint), F32))
    _, _, ox, oy, oz = lax.fori_loop(0, npoint, body, st)
    cx_ref[...] = ox
    cy_ref[...] = oy
    cz_ref[...] = oz


def _fps(x, y, z, npoint):
    B, N = x.shape
    out = jax.ShapeDtypeStruct((B, npoint), F32)
    return pl.pallas_call(
        functools.partial(_fps_body, npoint=npoint),
        out_shape=(out, out, out),
    )(x, y, z)


def _rank_of(mask_b, S, N):
    io = lax.broadcasted_iota(jnp.int32, (128, 128), 0)
    jo = lax.broadcasted_iota(jnp.int32, (128, 128), 1)
    tri = (io <= jo).astype(BF16)
    parts = []
    off = jnp.zeros((S, 1), F32)
    for c in range(N // 128):
        mc = mask_b[:, c * 128:(c + 1) * 128]
        rc = jnp.dot(mc, tri, preferred_element_type=F32) + off
        off = rc[:, 127:128]
        parts.append(rc)
    return jnp.concatenate(parts, axis=1)


def _sa_body(x_ref, y_ref, z_ref, cxc_ref, cyc_ref, czc_ref, val_ref,
             ptsT_ref, ctr_ref, pn1_ref, fn1_ref, w1p_ref, w1f_ref, b1_ref,
             g1_ref, be1_ref, w2_ref, b2_ref, g2_ref, be2_ref,
             w3_ref, b3_ref, g3_ref, be3_ref, out_ref,
             *, S, N, ns, r2, C):
    have_f = C > 0
    x = x_ref[0]
    y = y_ref[0]
    z = z_ref[0]
    cx = cxc_ref[...].reshape(S, 1)
    cy = cyc_ref[...].reshape(S, 1)
    cz = czc_ref[...].reshape(S, 1)
    nc = cx * cx + cy * cy + cz * cz
    npt = x * x + y * y + z * z
    ctr = ctr_ref[0]
    M = jnp.dot(ctr.astype(BF16), ptsT_ref[0],
                preferred_element_type=F32)
    sq = (nc + npt) - 2.0 * M
    mask_f = jnp.where(sq > r2, 0.0, 1.0)
    rank = _rank_of(mask_f.astype(BF16), S, N)
    empty = rank[:, N - 1:N] < 0.5
    val = val_ref[0]
    pn1 = pn1_ref[0]
    fn1 = fn1_ref[0] if have_f else None

    w1p = w1p_ref[...]
    w1f = w1f_ref[...] if have_f else None
    b1 = b1_ref[...]
    g1 = g1_ref[...]
    be1 = be1_ref[...]
    w2 = w2_ref[...]
    b2 = b2_ref[...]
    g2 = g2_ref[...]
    be2 = be2_ref[...]
    w3 = w3_ref[...]
    b3 = b3_ref[...]
    g3 = g3_ref[...]
    be3 = be3_ref[...]

    def gather_k(kf):
        oh = jnp.where(rank == kf, mask_f, 0.0).astype(BF16)
        r = jnp.dot(oh, val, preferred_element_type=F32)
        p12 = r[:, 3 * C:]
        rel = ((p12[:, :4] + p12[:, 4:8]) + p12[:, 8:]) - ctr
        fk = ((r[:, :C] + r[:, C:2 * C]) + r[:, 2 * C:3 * C]
              ) if have_f else None
        return rel, fk

    def relu(v):
        return jnp.maximum(v, 0.0)

    def mlp(rel, fk):
        h = jnp.dot(rel.astype(BF16), w1p, preferred_element_type=F32) + b1
        if have_f:
            h = h + jnp.dot(fk.astype(BF16), w1f, preferred_element_type=F32)
        h = relu(_bnorm(h, g1, be1))
        h = relu(_bnorm(
            jnp.dot(h.astype(BF16), w2, preferred_element_type=F32) + b2,
            g2, be2))
        h = relu(_bnorm(
            jnp.dot(h.astype(BF16), w3, preferred_element_type=F32) + b3,
            g3, be3))
        return h

    rel_g, f_g = gather_k(1.0)
    rel0 = jnp.where(empty, pn1 - ctr, rel_g)
    if have_f:
        f0 = jnp.where(empty, fn1, f_g)
    else:
        f0 = None
    acc0 = mlp(rel0, f0)

    def body(k, st):
        if have_f:
            acc, rel0, f0 = st
        else:
            acc, rel0 = st
            f0 = None
        kf = (k + 1).astype(F32)
        rel, fk = gather_k(kf)
        v = rel[:, 3:4] > 0.5
        rel = jnp.where(v, rel, rel0)
        if have_f:
            fk = jnp.where(v, fk, f0)
        acc = jnp.maximum(acc, mlp(rel, fk))
        return (acc, rel0, f0) if have_f else (acc, rel0)

    st = (acc0, rel0, f0) if have_f else (acc0, rel0)
    st = lax.fori_loop(1, ns, body, st)
    out_ref[0] = st[0]


def _drop_f(body):
    def wrapped(x, y, z, cxc, cyc, czc, val, ptsT, ctr, pn1,
                w1p, b1, g1, be1, w2, b2, g2, be2, w3, b3, g3, be3, out):
        return body(x, y, z, cxc, cyc, czc, val, ptsT, ctr, pn1, None,
                    w1p, None, b1, g1, be1, w2, b2, g2, be2,
                    w3, b3, g3, be3, out)
    return wrapped


def _hilo(v):
    hi = v.astype(BF16)
    r = v - hi.astype(F32)
    mid = r.astype(BF16)
    lo = (r - mid.astype(F32)).astype(BF16)
    return hi, mid, lo


def _sa_level(x, y, z, ncx, ncy, ncz, feats, layers, ns, radius):
    B, N = x.shape
    S = ncx.shape[1]
    (W1, b1, g1, be1), (W2, b2, g2, be2), (W3, b3, g3, be3) = layers
    C3 = W3.shape[1]
    one = jnp.ones_like(x)
    zero = jnp.zeros_like(x)
    xh, xm, xl = _hilo(x)
    yh, ym, yl = _hilo(y)
    zh, zm, zl = _hilo(z)
    ob = one.astype(BF16)
    zb = zero.astype(BF16)
    pts12 = jnp.stack([xh, yh, zh, ob, xm, ym, zm, zb,
                       xl, yl, zl, zb], -1)
    ptsT = jnp.stack([x, y, z, one], axis=1).astype(BF16)
    ctr = jnp.stack([ncx, ncy, ncz, jnp.zeros_like(ncx)], -1)
    pn1 = jnp.stack([x[:, N - 1:], y[:, N - 1:], z[:, N - 1:],
                     one[:, N - 1:]], -1)
    cxc = ncx[..., None]
    cyc = ncy[..., None]
    czc = ncz[..., None]
    w1p = jnp.concatenate(
        [W1[:3], jnp.zeros((1, W1.shape[1]), F32)], 0).astype(BF16)
    have_f = feats is not None
    if have_f:
        C = feats.shape[2]
        fh, fm, fl = _hilo(feats)
        val = jnp.concatenate([fh, fm, fl, pts12], -1)
        fn1 = feats[:, N - 1:, :]
        w1f = W1[3:].astype(BF16)
    else:
        C = 0
        val = pts12
        fn1 = None
        w1f = None

    def row(v):
        return v.reshape(1, -1)

    body = functools.partial(_sa_body, S=S, N=N, ns=ns, r2=radius * radius,
                             C=C)
    if not have_f:
        body = _drop_f(body)

    in_specs = [
        pl.BlockSpec((1, 1, N), lambda b: (b, 0, 0)),
        pl.BlockSpec((1, 1, N), lambda b: (b, 0, 0)),
        pl.BlockSpec((1, 1, N), lambda b: (b, 0, 0)),
        pl.BlockSpec((1, S, 1), lambda b: (b, 0, 0)),
        pl.BlockSpec((1, S, 1), lambda b: (b, 0, 0)),
        pl.BlockSpec((1, S, 1), lambda b: (b, 0, 0)),
        pl.BlockSpec((1, N, 3 * C + 12), lambda b: (b, 0, 0)),
        pl.BlockSpec((1, 4, N), lambda b: (b, 0, 0)),
        pl.BlockSpec((1, S, 4), lambda b: (b, 0, 0)),
        pl.BlockSpec((1, 1, 4), lambda b: (b, 0, 0)),
    ]
    args = [x[:, None, :], y[:, None, :], z[:, None, :], cxc, cyc, czc,
            val, ptsT, ctr, pn1]
    if have_f:
        in_specs.append(pl.BlockSpec((1, 1, C), lambda b: (b, 0, 0)))
        args.append(fn1)
    wspecs = []
    wargs = []
    for w in (w1p, w1f, row(b1), row(g1), row(be1),
              W2.astype(BF16), row(b2), row(g2), row(be2),
              W3.astype(BF16), row(b3), row(g3), row(be3)):
        if not have_f and w is w1f:
            continue
        wspecs.append(pl.BlockSpec(w.shape, lambda b, nd=w.ndim: (0,) * nd))
        wargs.append(w)

    return pl.pallas_call(
        body,
        grid=(B,),
        in_specs=in_specs + wspecs,
        out_specs=pl.BlockSpec((1, S, C3), lambda b: (b, 0, 0)),
        out_shape=jax.ShapeDtypeStruct((B, S, C3), F32),
    )(*args, *wargs)


def _sa3_body(c_ref, f_ref, w1p_ref, w1f_ref, b1_ref, g1_ref, be1_ref,
              w2_ref, b2_ref, g2_ref, be2_ref,
              w3_ref, b3_ref, g3_ref, be3_ref, out_ref, *, nb, P):
    c = c_ref[...].reshape(nb * P, 4)
    f = f_ref[...].reshape(nb * P, -1)

    def relu(v):
        return jnp.maximum(v, 0.0)

    h = _mm(c, w1p_ref[...]) + _mm(f, w1f_ref[...]) + b1_ref[...]
    h = relu(_bnorm(h, g1_ref[...], be1_ref[...]))
    h = relu(_bnorm(_mm(h, w2_ref[...]) + b2_ref[...],
                    g2_ref[...], be2_ref[...]))
    h = relu(_bnorm(_mm(h, w3_ref[...]) + b3_ref[...],
                    g3_ref[...], be3_ref[...]))
    C3 = h.shape[1]
    out_ref[...] = jnp.max(h.reshape(nb, P, C3), axis=1)


def _sa3(cx, cy, cz, feats, layers, nb=8):
    B, P = cx.shape
    (W1, b1, g1, be1), (W2, b2, g2, be2), (W3, b3, g3, be3) = layers
    C = feats.shape[2]
    C3 = W3.shape[1]
    c4 = jnp.stack([cx, cy, cz, jnp.zeros_like(cx)], -1)
    w1p = jnp.concatenate([W1[:3], jnp.zeros((1, W1.shape[1]), F32)], 0)
    w1f = W1[3:]

    def row(v):
        return v.reshape(1, -1)

    wlist = [w1p, w1f, row(b1), row(g1), row(be1), W2, row(b2), row(g2),
             row(be2), W3, row(b3), row(g3), row(be3)]
    wspecs = [pl.BlockSpec(w.shape, lambda b, nd=w.ndim: (0,) * nd)
              for w in wlist]
    return pl.pallas_call(
        functools.partial(_sa3_body, nb=nb, P=P),
        grid=(B // nb,),
        in_specs=[pl.BlockSpec((nb, P, 4), lambda b: (b, 0, 0)),
                  pl.BlockSpec((nb, P, C), lambda b: (b, 0, 0))] + wspecs,
        out_specs=pl.BlockSpec((nb, C3), lambda b: (b, 0)),
        out_shape=jax.ShapeDtypeStruct((B, C3), F32),
    )(c4, feats, *wlist)


def _head_body(x_ref, w1_ref, b1_ref, g1_ref, be1_ref,
               w2_ref, b2_ref, g2_ref, be2_ref, w3_ref, b3_ref, out_ref):
    def relu(v):
        return jnp.maximum(v, 0.0)

    h = relu(_bnorm(_mm(x_ref[...], w1_ref[...]) + b1_ref[...],
                    g1_ref[...], be1_ref[...]))
    h = relu(_bnorm(_mm(h, w2_ref[...]) + b2_ref[...],
                    g2_ref[...], be2_ref[...]))
    out_ref[...] = _mm(h, w3_ref[...]) + b3_ref[...]


def _head(x, head):
    (W1, b1, g1, be1, W2, b2, g2, be2, W3, b3) = head
    B = x.shape[0]

    def row(v):
        return v.reshape(1, -1)

    args = [x, W1, row(b1), row(g1), row(be1),
            W2, row(b2), row(g2), row(be2), W3, row(b3)]
    return pl.pallas_call(
        _head_body,
        out_shape=jax.ShapeDtypeStruct((B, W3.shape[1]), F32),
    )(*args)


def kernel(points, sa1, sa2, sa3, head):
    x = points[..., 0]
    y = points[..., 1]
    z = points[..., 2]
    c1x, c1y, c1z = _fps(x, y, z, 512)
    f1 = _sa_level(x, y, z, c1x, c1y, c1z, None, sa1, 32, 0.2)
    c2x, c2y, c2z = _fps(c1x, c1y, c1z, 128)
    f2 = _sa_level(c1x, c1y, c1z, c2x, c2y, c2z, f1, sa2, 64, 0.4)
    f3 = _sa3(c2x, c2y, c2z, f2, sa3)
    return _head(f3, head)
